# Initial kernel scaffold; baseline (speedup 1.0000x reference)
import functools

import jax
import jax.numpy as jnp
from jax import lax
from jax.experimental import pallas as pl
from jax.experimental.pallas import tpu as pltpu
from jax.experimental.pallas import tpu_sc as plsc

ALPHA = 0.1
K_HOPS = 10

NC = 2
NS = 16
NW = NC * NS
LANES = 16
EB = 128


def _sc_propagate(h, src, dst, val, zeros, n_nodes, d, blocks_per_worker):
    per_w = blocks_per_worker * EB

    rows_main = (n_nodes // NS) & ~7
    rem = n_nodes - rows_main * NS

    mesh = plsc.VectorSubcoreMesh(core_axis_name="c", subcore_axis_name="s")

    @functools.partial(
        pl.kernel,
        out_type=jax.ShapeDtypeStruct((NC * n_nodes, d), jnp.float32),
        mesh=mesh,
        scratch_types=[
            pltpu.VMEM((EB,), jnp.int32),
            pltpu.VMEM((EB,), jnp.int32),
            pltpu.VMEM((EB,), jnp.float32),
            pltpu.VMEM((EB, d), jnp.float32),
            pltpu.VMEM_SHARED((n_nodes, d), jnp.float32),
            pltpu.SemaphoreType.DMA,
        ],
    )
    def prop(h_hbm, src_hbm, dst_hbm, val_hbm, zero_hbm, out_hbm,
             src_v, dst_v, val_v, rows_v, acc_sh, sem):
        cid = lax.axis_index("c")
        sid = lax.axis_index("s")
        wid = cid * NS + sid

        r0 = sid * rows_main
        pltpu.sync_copy(zero_hbm.at[pl.ds(r0, rows_main)],
                        acc_sh.at[pl.ds(r0, rows_main)])
        if rem:
            @pl.when(sid == NS - 1)
            def _():
                pltpu.sync_copy(zero_hbm.at[pl.ds(rows_main * NS, rem)],
                                acc_sh.at[pl.ds(rows_main * NS, rem)])
        plsc.subcore_barrier()

        base = wid * per_w

        @pl.loop(0, blocks_per_worker)
        def _(b):
            off = base + b * EB
            pltpu.sync_copy(src_hbm.at[pl.ds(off, EB)], src_v)
            pltpu.sync_copy(dst_hbm.at[pl.ds(off, EB)], dst_v)
            pltpu.sync_copy(val_hbm.at[pl.ds(off, EB)], val_v)
            pltpu.async_copy(h_hbm.at[src_v], rows_v, sem).wait()

            @pl.loop(0, EB)
            def _(r):
                vv = plsc.load_gather(
                    val_v, [jnp.full((LANES,), r, dtype=jnp.int32)])
                for c in range(d // LANES):
                    sl = pl.ds(c * LANES, LANES)
                    rows_v[r, sl] = rows_v[r, sl] * vv

            pltpu.sync_copy(rows_v, acc_sh.at[dst_v], add=True)

        plsc.subcore_barrier()

        o0 = cid * n_nodes + r0
        pltpu.sync_copy(acc_sh.at[pl.ds(r0, rows_main)],
                        out_hbm.at[pl.ds(o0, rows_main)])
        if rem:
            @pl.when(sid == NS - 1)
            def _():
                pltpu.sync_copy(
                    acc_sh.at[pl.ds(rows_main * NS, rem)],
                    out_hbm.at[pl.ds(cid * n_nodes + rows_main * NS, rem)])

    return prop(h, src, dst, val, zeros)


def _tc_combine(p, x, n_nodes, d):
    def body(p_ref, x_ref, o_ref):
        agg = p_ref[0:n_nodes, :] + p_ref[n_nodes:2 * n_nodes, :]
        o_ref[...] = (1.0 - ALPHA) * agg + ALPHA * x_ref[...]

    return pl.pallas_call(
        body,
        out_shape=jax.ShapeDtypeStruct((n_nodes, d), jnp.float32),
    )(p, x)


def kernel(x, edge_index, adj_values):
    n_nodes, d = x.shape
    dst = edge_index[0]
    src = edge_index[1]
    e = dst.shape[0]

    blocks_per_worker = -(-e // (NW * EB))
    e_pad = blocks_per_worker * EB * NW
    pad = e_pad - e
    if pad:
        src = jnp.concatenate([src, jnp.zeros((pad,), src.dtype)])
        dst = jnp.concatenate([dst, jnp.zeros((pad,), dst.dtype)])
        adj = jnp.concatenate([adj_values, jnp.zeros((pad,), adj_values.dtype)])
    else:
        adj = adj_values
    zeros = jnp.zeros((n_nodes, d), jnp.float32)

    h = x
    for _ in range(K_HOPS):
        p = _sc_propagate(h, src, dst, adj, zeros, n_nodes, d,
                          blocks_per_worker)
        h = _tc_combine(p, x, n_nodes, d)
    return h


# speedup vs baseline: 3.1775x; 3.1775x over previous
import dataclasses
import functools

import jax
import jax.numpy as jnp
from jax import lax
from jax.experimental import pallas as pl
from jax.experimental.pallas import tpu as pltpu
from jax.experimental.pallas import tpu_sc as plsc

ALPHA = 0.1
K_HOPS = 10

NC = 2
NS = 16
NW = NC * NS
LANES = 16
EB = 128


def _sc_propagate(h, src, dst, val, zeros, n_nodes, d, blocks_per_worker):
    per_w = blocks_per_worker * EB

    rows_main = (n_nodes // NS) & ~7
    rem = n_nodes - rows_main * NS

    mesh = plsc.VectorSubcoreMesh(core_axis_name="c", subcore_axis_name="s")

    cp = pltpu.CompilerParams()
    if "needs_layout_passes" in pltpu.CompilerParams.__dataclass_fields__:
        cp = dataclasses.replace(cp, needs_layout_passes=False)

    @functools.partial(
        pl.kernel,
        out_type=jax.ShapeDtypeStruct((NC * n_nodes, d), jnp.float32),
        mesh=mesh,
        compiler_params=cp,
        scratch_types=[
            pltpu.VMEM((EB,), jnp.int32),
            pltpu.VMEM((EB,), jnp.int32),
            pltpu.VMEM((EB,), jnp.float32),
            pltpu.VMEM((EB, d), jnp.float32),
            pltpu.VMEM_SHARED((n_nodes, d), jnp.float32),
            pltpu.SemaphoreType.DMA,
        ],
    )
    def prop(h_hbm, src_hbm, dst_hbm, val_hbm, zero_hbm, out_hbm,
             src_v, dst_v, val_v, rows_v, acc_sh, sem):
        cid = lax.axis_index("c")
        sid = lax.axis_index("s")
        wid = cid * NS + sid

        r0 = sid * rows_main
        pltpu.sync_copy(zero_hbm.at[pl.ds(r0, rows_main)],
                        acc_sh.at[pl.ds(r0, rows_main)])
        if rem:
            @pl.when(sid == NS - 1)
            def _():
                pltpu.sync_copy(zero_hbm.at[pl.ds(rows_main * NS, rem)],
                                acc_sh.at[pl.ds(rows_main * NS, rem)])
        plsc.subcore_barrier()

        base = wid * per_w

        @pl.loop(0, blocks_per_worker)
        def _(b):
            off = base + b * EB
            pltpu.sync_copy(src_hbm.at[pl.ds(off, EB)], src_v)
            pltpu.sync_copy(dst_hbm.at[pl.ds(off, EB)], dst_v)
            pltpu.sync_copy(val_hbm.at[pl.ds(off, EB)], val_v)
            pltpu.async_copy(h_hbm.at[src_v], rows_v, sem).wait()

            @pl.loop(0, EB)
            def _(r):
                vv = plsc.load_gather(
                    val_v, [jnp.full((LANES,), r, dtype=jnp.int32)])
                for c in range(d // LANES):
                    sl = pl.ds(c * LANES, LANES)
                    rows_v[r, sl] = rows_v[r, sl] * vv

            pltpu.sync_copy(rows_v, acc_sh.at[dst_v], add=True)

        plsc.subcore_barrier()

        o0 = cid * n_nodes + r0
        pltpu.sync_copy(acc_sh.at[pl.ds(r0, rows_main)],
                        out_hbm.at[pl.ds(o0, rows_main)])
        if rem:
            @pl.when(sid == NS - 1)
            def _():
                pltpu.sync_copy(
                    acc_sh.at[pl.ds(rows_main * NS, rem)],
                    out_hbm.at[pl.ds(cid * n_nodes + rows_main * NS, rem)])

    return prop(h, src, dst, val, zeros)


def _tc_combine(p, x, n_nodes, d):
    def body(p_ref, x_ref, o_ref):
        agg = p_ref[0:n_nodes, :] + p_ref[n_nodes:2 * n_nodes, :]
        o_ref[...] = (1.0 - ALPHA) * agg + ALPHA * x_ref[...]

    return pl.pallas_call(
        body,
        out_shape=jax.ShapeDtypeStruct((n_nodes, d), jnp.float32),
    )(p, x)


def kernel(x, edge_index, adj_values):
    n_nodes, d = x.shape
    dst = edge_index[0]
    src = edge_index[1]
    e = dst.shape[0]

    blocks_per_worker = -(-e // (NW * EB))
    e_pad = blocks_per_worker * EB * NW
    pad = e_pad - e
    if pad:
        src = jnp.concatenate([src, jnp.zeros((pad,), src.dtype)])
        dst = jnp.concatenate([dst, jnp.zeros((pad,), dst.dtype)])
        adj = jnp.concatenate([adj_values, jnp.zeros((pad,), adj_values.dtype)])
    else:
        adj = adj_values
    zeros = jnp.zeros((n_nodes, d), jnp.float32)

    h = x
    for _ in range(K_HOPS):
        p = _sc_propagate(h, src, dst, adj, zeros, n_nodes, d,
                          blocks_per_worker)
        h = _tc_combine(p, x, n_nodes, d)
    return h


# speedup vs baseline: 5.1059x; 1.6069x over previous
import dataclasses
import functools

import jax
import jax.numpy as jnp
from jax import lax
from jax.experimental import pallas as pl
from jax.experimental.pallas import tpu as pltpu
from jax.experimental.pallas import tpu_sc as plsc

ALPHA = 0.1
K_HOPS = 10

NC = 2
NS = 16
LANES = 16
EB = 128


def _sc_propagate(h2, src3, dst3, val3, zeros, n_nodes, dh, nb):
    rows_main = (n_nodes // NS) & ~7
    rem = n_nodes - rows_main * NS

    mesh = plsc.VectorSubcoreMesh(core_axis_name="c", subcore_axis_name="s")

    cp = pltpu.CompilerParams()
    fields = pltpu.CompilerParams.__dataclass_fields__
    if "needs_layout_passes" in fields:
        cp = dataclasses.replace(cp, needs_layout_passes=False)
    if "use_tc_tiling_on_sc" in fields:
        cp = dataclasses.replace(cp, use_tc_tiling_on_sc=False)

    @functools.partial(
        pl.kernel,
        out_type=jax.ShapeDtypeStruct((NC * n_nodes, dh), jnp.float32),
        mesh=mesh,
        compiler_params=cp,
        scratch_types=[
            pltpu.VMEM((nb, EB), jnp.int32),
            pltpu.VMEM((nb, EB), jnp.int32),
            pltpu.VMEM((nb, EB), jnp.float32),
            pltpu.VMEM((2, EB, dh), jnp.float32),
            pltpu.VMEM_SHARED((n_nodes, dh), jnp.float32),
            pltpu.SemaphoreType.DMA,
            pltpu.SemaphoreType.DMA,
            pltpu.SemaphoreType.DMA,
        ],
    )
    def prop(h_hbm, src_hbm, dst_hbm, val_hbm, zero_hbm, out_hbm,
             src_all, dst_all, val_all, rows_v, acc_sh, sem_i, sem_g0, sem_g1):
        cid = lax.axis_index("c")
        sid = lax.axis_index("s")
        sem_g = (sem_g0, sem_g1)

        pltpu.async_copy(src_hbm.at[sid], src_all, sem_i)
        pltpu.async_copy(dst_hbm.at[sid], dst_all, sem_i)
        pltpu.async_copy(val_hbm.at[sid], val_all, sem_i)

        r0 = sid * rows_main
        pltpu.sync_copy(zero_hbm.at[pl.ds(r0, rows_main)],
                        acc_sh.at[pl.ds(r0, rows_main)])
        if rem:
            @pl.when(sid == NS - 1)
            def _():
                pltpu.sync_copy(zero_hbm.at[pl.ds(rows_main * NS, rem)],
                                acc_sh.at[pl.ds(rows_main * NS, rem)])

        pltpu.make_async_copy(src_hbm.at[sid], src_all, sem_i).wait()
        pltpu.make_async_copy(dst_hbm.at[sid], dst_all, sem_i).wait()
        pltpu.make_async_copy(val_hbm.at[sid], val_all, sem_i).wait()

        bias = cid * n_nodes

        @pl.loop(0, nb)
        def _(b):
            for c in range(EB // LANES):
                sl = pl.ds(c * LANES, LANES)
                src_all[b, sl] = src_all[b, sl] + bias

        pltpu.async_copy(h_hbm.at[src_all.at[0]], rows_v.at[0], sem_g0)

        plsc.subcore_barrier()

        def substep(k, p):
            q = 1 - p
            pltpu.make_async_copy(
                h_hbm.at[src_all.at[k]], rows_v.at[p], sem_g[p]).wait()

            @pl.when(k + 1 < nb)
            def _():
                pltpu.async_copy(
                    h_hbm.at[src_all.at[k + 1]], rows_v.at[q], sem_g[q])

            @pl.loop(0, EB)
            def _(r):
                vv = plsc.load_gather(
                    val_all, [jnp.full((LANES,), k, dtype=jnp.int32),
                              jnp.full((LANES,), r, dtype=jnp.int32)])
                for c in range(dh // LANES):
                    sl = pl.ds(c * LANES, LANES)
                    rows_v[p, r, sl] = rows_v[p, r, sl] * vv

            pltpu.sync_copy(rows_v.at[p], acc_sh.at[dst_all.at[k]], add=True)

        @pl.loop(0, nb // 2)
        def _(i):
            substep(2 * i, 0)
            substep(2 * i + 1, 1)

        plsc.subcore_barrier()

        o0 = cid * n_nodes + r0
        pltpu.sync_copy(acc_sh.at[pl.ds(r0, rows_main)],
                        out_hbm.at[pl.ds(o0, rows_main)])
        if rem:
            @pl.when(sid == NS - 1)
            def _():
                pltpu.sync_copy(
                    acc_sh.at[pl.ds(rows_main * NS, rem)],
                    out_hbm.at[pl.ds(cid * n_nodes + rows_main * NS, rem)])

    return prop(h2, src3, dst3, val3, zeros)


def _tc_combine(agg, x2, n2, dh):
    def body(a_ref, x_ref, o_ref):
        o_ref[...] = (1.0 - ALPHA) * a_ref[...] + ALPHA * x_ref[...]

    return pl.pallas_call(
        body,
        out_shape=jax.ShapeDtypeStruct((n2, dh), jnp.float32),
    )(agg, x2)


def _tc_combine_final(agg, x, n_nodes, d, dh):
    def body(a_ref, x_ref, o_ref):
        for c in range(2):
            sl = pl.ds(c * dh, dh)
            o_ref[:, sl] = ((1.0 - ALPHA)
                            * a_ref[c * n_nodes:(c + 1) * n_nodes, :]
                            + ALPHA * x_ref[:, sl])

    return pl.pallas_call(
        body,
        out_shape=jax.ShapeDtypeStruct((n_nodes, d), jnp.float32),
    )(agg, x)


def kernel(x, edge_index, adj_values):
    n_nodes, d = x.shape
    dh = d // NC
    dst = edge_index[0]
    src = edge_index[1]
    e = dst.shape[0]

    nb = -(-e // (NS * EB))
    nb += nb % 2
    e_pad = nb * EB * NS
    pad = e_pad - e
    if pad:
        src = jnp.concatenate([src, jnp.zeros((pad,), src.dtype)])
        dst = jnp.concatenate([dst, jnp.zeros((pad,), dst.dtype)])
        adj = jnp.concatenate([adj_values, jnp.zeros((pad,), adj_values.dtype)])
    else:
        adj = adj_values
    src3 = src.reshape(NS, nb, EB)
    dst3 = dst.reshape(NS, nb, EB)
    val3 = adj.reshape(NS, nb, EB)
    zeros = jnp.zeros((n_nodes, dh), jnp.float32)

    x2 = jnp.concatenate([x[:, :dh], x[:, dh:]], axis=0)

    h2 = x2
    for hop in range(K_HOPS):
        agg = _sc_propagate(h2, src3, dst3, val3, zeros, n_nodes, dh, nb)
        if hop < K_HOPS - 1:
            h2 = _tc_combine(agg, x2, NC * n_nodes, dh)
        else:
            return _tc_combine_final(agg, x, n_nodes, d, dh)


# speedup vs baseline: 5.2906x; 1.0362x over previous
import dataclasses
import functools

import jax
import jax.numpy as jnp
from jax import lax
from jax.experimental import pallas as pl
from jax.experimental.pallas import tpu as pltpu
from jax.experimental.pallas import tpu_sc as plsc

ALPHA = 0.1
K_HOPS = 10

NC = 2
NS = 16
LANES = 16
EB = 128
CH = 104


def _sc_hop(h2, src4, dst3, val3, zeros, x2, n_nodes, dh, nb):
    rows_main = (n_nodes // NS) & ~7
    rem = n_nodes - rows_main * NS
    n_ch = rows_main // CH
    assert n_ch * CH == rows_main and CH <= EB and rem <= EB

    mesh = plsc.VectorSubcoreMesh(core_axis_name="c", subcore_axis_name="s")

    cp = pltpu.CompilerParams()
    fields = pltpu.CompilerParams.__dataclass_fields__
    if "needs_layout_passes" in fields:
        cp = dataclasses.replace(cp, needs_layout_passes=False)
    if "use_tc_tiling_on_sc" in fields:
        cp = dataclasses.replace(cp, use_tc_tiling_on_sc=False)

    @functools.partial(
        pl.kernel,
        out_type=jax.ShapeDtypeStruct((NC * n_nodes, dh), jnp.float32),
        mesh=mesh,
        compiler_params=cp,
        scratch_types=[
            pltpu.VMEM((nb, EB), jnp.int32),
            pltpu.VMEM((nb, EB), jnp.int32),
            pltpu.VMEM((nb, EB), jnp.float32),
            pltpu.VMEM((2, EB, dh), jnp.float32),
            pltpu.VMEM_SHARED((n_nodes, dh), jnp.float32),
            pltpu.SemaphoreType.DMA,
            pltpu.SemaphoreType.DMA,
            pltpu.SemaphoreType.DMA,
        ],
    )
    def prop(h_hbm, src_hbm, dst_hbm, val_hbm, zero_hbm, x2_hbm, out_hbm,
             src_all, dst_all, val_all, rows_v, acc_sh, sem_i, sem_g0, sem_g1):
        cid = lax.axis_index("c")
        sid = lax.axis_index("s")
        wid = cid * NS + sid
        sem_g = (sem_g0, sem_g1)

        pltpu.async_copy(src_hbm.at[wid], src_all, sem_i)
        pltpu.async_copy(dst_hbm.at[sid], dst_all, sem_i)
        pltpu.async_copy(val_hbm.at[sid], val_all, sem_i)

        r0 = sid * rows_main
        pltpu.sync_copy(zero_hbm.at[pl.ds(r0, rows_main)],
                        acc_sh.at[pl.ds(r0, rows_main)])
        if rem:
            @pl.when(sid == NS - 1)
            def _():
                pltpu.sync_copy(zero_hbm.at[pl.ds(rows_main * NS, rem)],
                                acc_sh.at[pl.ds(rows_main * NS, rem)])

        pltpu.make_async_copy(src_hbm.at[wid], src_all, sem_i).wait()
        pltpu.make_async_copy(dst_hbm.at[sid], dst_all, sem_i).wait()
        pltpu.make_async_copy(val_hbm.at[sid], val_all, sem_i).wait()

        pltpu.async_copy(h_hbm.at[src_all.at[0]], rows_v.at[0], sem_g0)

        plsc.subcore_barrier()

        def substep(k, p):
            q = 1 - p
            pltpu.make_async_copy(
                h_hbm.at[src_all.at[k]], rows_v.at[p], sem_g[p]).wait()

            @pl.when(k + 1 < nb)
            def _():
                pltpu.async_copy(
                    h_hbm.at[src_all.at[k + 1]], rows_v.at[q], sem_g[q])

            @pl.loop(0, EB)
            def _(r):
                vv = plsc.load_gather(
                    val_all, [jnp.full((LANES,), k, dtype=jnp.int32),
                              jnp.full((LANES,), r, dtype=jnp.int32)])
                for c in range(dh // LANES):
                    sl = pl.ds(c * LANES, LANES)
                    rows_v[p, r, sl] = rows_v[p, r, sl] * vv

            pltpu.sync_copy(rows_v.at[p], acc_sh.at[dst_all.at[k]], add=True)

        @pl.loop(0, nb // 2)
        def _(i):
            substep(2 * i, 0)
            substep(2 * i + 1, 1)

        plsc.subcore_barrier()

        def combine_rows(row0, nrows):
            a_v = rows_v.at[0, pl.ds(0, nrows)]
            x_v = rows_v.at[1, pl.ds(0, nrows)]
            pltpu.sync_copy(acc_sh.at[pl.ds(row0, nrows)], a_v)
            pltpu.sync_copy(x2_hbm.at[pl.ds(cid * n_nodes + row0, nrows)], x_v)

            @pl.loop(0, nrows)
            def _(r):
                for c in range(dh // LANES):
                    sl = pl.ds(c * LANES, LANES)
                    rows_v[0, r, sl] = ((1.0 - ALPHA) * rows_v[0, r, sl]
                                        + ALPHA * rows_v[1, r, sl])

            pltpu.sync_copy(
                a_v, out_hbm.at[pl.ds(cid * n_nodes + row0, nrows)])

        @pl.loop(0, n_ch)
        def _(j):
            combine_rows(r0 + j * CH, CH)

        if rem:
            @pl.when(sid == NS - 1)
            def _():
                combine_rows(rows_main * NS, rem)

    return prop(h2, src4, dst3, val3, zeros, x2)


def kernel(x, edge_index, adj_values):
    n_nodes, d = x.shape
    dh = d // NC
    dst = edge_index[0]
    src = edge_index[1]
    e = dst.shape[0]

    nb = -(-e // (NS * EB))
    nb += nb % 2
    e_pad = nb * EB * NS
    pad = e_pad - e
    if pad:
        src = jnp.concatenate([src, jnp.zeros((pad,), src.dtype)])
        dst = jnp.concatenate([dst, jnp.zeros((pad,), dst.dtype)])
        adj = jnp.concatenate([adj_values, jnp.zeros((pad,), adj_values.dtype)])
    else:
        adj = adj_values
    src3 = src.reshape(NS, nb, EB)
    src4 = jnp.concatenate([src3, src3 + n_nodes], axis=0)
    dst3 = dst.reshape(NS, nb, EB)
    val3 = adj.reshape(NS, nb, EB)
    zeros = jnp.zeros((n_nodes, dh), jnp.float32)

    x2 = jnp.concatenate([x[:, :dh], x[:, dh:]], axis=0)

    h2 = x2
    for _ in range(K_HOPS):
        h2 = _sc_hop(h2, src4, dst3, val3, zeros, x2, n_nodes, dh, nb)

    return jnp.concatenate([h2[:n_nodes], h2[n_nodes:]], axis=1)
